# baseline (speedup 1.0000x reference)
import jax
import jax.numpy as jnp
from jax.experimental import pallas as pl
from jax.experimental.pallas import tpu as pltpu

_BM = 256
_NBUF = 4


def _gcn8(x_ref, w_ref, b_ref, adj_hbm, out_ref, support_ref, buf_ref, sem_ref):
    i = pl.program_id(0)
    n_cells = pl.num_programs(0)

    def slab_copy(block, slot):
        return pltpu.make_async_copy(
            adj_hbm.at[pl.ds(block * _BM, _BM), :],
            buf_ref.at[slot],
            sem_ref.at[slot],
        )

    @pl.when(i == 0)
    def _prologue():
        for s in range(_NBUF):
            slab_copy(s, s).start()
        support_ref[...] = jnp.dot(
            x_ref[...], w_ref[...], preferred_element_type=jnp.float32
        )

    slot = jax.lax.rem(i, _NBUF)
    slab_copy(i, slot).wait()
    acc = jnp.dot(
        buf_ref[slot], support_ref[...], preferred_element_type=jnp.float32
    )
    x_blk = x_ref[pl.ds(i * _BM, _BM), :]
    out_ref[...] = jnp.tanh(acc + b_ref[...] + x_blk)

    @pl.when(i + _NBUF < n_cells)
    def _prefetch():
        slab_copy(i + _NBUF, slot).start()


def kernel(x, adj, W, b):
    n, d = x.shape
    b2 = b.reshape(1, d)
    return pl.pallas_call(
        _gcn8,
        grid=(n // _BM,),
        in_specs=[
            pl.BlockSpec((n, d), lambda i: (0, 0)),
            pl.BlockSpec((d, d), lambda i: (0, 0)),
            pl.BlockSpec((1, d), lambda i: (0, 0)),
            pl.BlockSpec(memory_space=pltpu.MemorySpace.HBM),
        ],
        out_specs=pl.BlockSpec((_BM, d), lambda i: (i, 0)),
        out_shape=jax.ShapeDtypeStruct((n, d), jnp.float32),
        scratch_shapes=[
            pltpu.VMEM((n, d), jnp.float32),
            pltpu.VMEM((_NBUF, _BM, n), jnp.float32),
            pltpu.SemaphoreType.DMA((_NBUF,)),
        ],
        compiler_params=pltpu.CompilerParams(
            dimension_semantics=("arbitrary",),
        ),
    )(x, W, b2, adj)


# speedup vs baseline: 1.0474x; 1.0474x over previous
import jax
import jax.numpy as jnp
from jax.experimental import pallas as pl
from jax.experimental.pallas import tpu as pltpu

_BM = 256


def _gcn_block_kernel(x_ref, w_ref, b_ref, adj_ref, out_ref, support_ref):
    i = pl.program_id(0)

    @pl.when(i == 0)
    def _compute_support():
        support_ref[...] = jnp.dot(
            x_ref[...], w_ref[...], preferred_element_type=jnp.float32
        )

    acc = jnp.dot(
        adj_ref[...], support_ref[...], preferred_element_type=jnp.float32
    )
    x_blk = x_ref[pl.ds(i * _BM, _BM), :]
    out_ref[...] = jnp.tanh(acc + b_ref[...] + x_blk)


def kernel(x, adj, W, b):
    n, d = x.shape
    b2 = b.reshape(1, d)
    return pl.pallas_call(
        _gcn_block_kernel,
        grid=(n // _BM,),
        in_specs=[
            pl.BlockSpec((n, d), lambda i: (0, 0)),
            pl.BlockSpec((d, d), lambda i: (0, 0)),
            pl.BlockSpec((1, d), lambda i: (0, 0)),
            pl.BlockSpec((_BM, n), lambda i: (i, 0)),
        ],
        out_specs=pl.BlockSpec((_BM, d), lambda i: (i, 0)),
        out_shape=jax.ShapeDtypeStruct((n, d), jnp.float32),
        scratch_shapes=[pltpu.VMEM((n, d), jnp.float32)],
        compiler_params=pltpu.CompilerParams(
            dimension_semantics=("arbitrary",),
        ),
    )(x, W, b2, adj)
